# Initial kernel scaffold; baseline (speedup 1.0000x reference)
import jax
import jax.numpy as jnp
from jax.experimental import pallas as pl


def kernel(input, dim, index, src):
    raise NotImplementedError("write your pallas kernel here")


# speedup vs baseline: 4.2326x; 4.2326x over previous
import functools

import jax
import jax.numpy as jnp
from jax import lax
from jax.experimental import pallas as pl
from jax.experimental.pallas import tpu as pltpu
from jax.experimental.pallas import tpu_sc as plsc


def _build(M, D, B):
  NC, NS, L = 2, 16, 16
  CPS = D // NC
  CW = CPS // NS
  SH = (CPS - 1).bit_length()
  SPC = M * CPS
  CH = 40000
  IC = 1024
  V = IC // L

  mesh = plsc.VectorSubcoreMesh(core_axis_name="c", subcore_axis_name="s")

  @functools.partial(
      pl.kernel,
      out_type=jax.ShapeDtypeStruct((NC * SPC,), jnp.float32),
      mesh=mesh,
      scratch_types=[
          pltpu.VMEM((CH,), jnp.float32),
          pltpu.VMEM((CW, IC), jnp.int32),
          pltpu.VMEM((CW, IC), jnp.float32),
          pltpu.VMEM((CW * IC,), jnp.int32),
          pltpu.VMEM((CW * IC,), jnp.float32),
          pltpu.SemaphoreType.DMA,
      ],
  )
  def k(inpS, idxT, srcT, w, copy_v, idx_v, src_v, offs_v, upd_v, sem):
    core = lax.axis_index("c")
    sub = lax.axis_index("s")
    base = core * SPC

    @pl.loop(sub, SPC // CH, step=NS)
    def _copy(t):
      o = base + t * CH
      pltpu.sync_copy(inpS.at[pl.ds(o, CH)], copy_v)
      pltpu.sync_copy(copy_v, w.at[pl.ds(o, CH)])

    plsc.subcore_barrier()

    c0 = core * CPS + sub * CW
    l0 = sub * CW

    @pl.loop(0, B // IC)
    def _chunk(t):
      i0 = t * IC
      pltpu.sync_copy(idxT.at[pl.ds(c0, CW), pl.ds(i0, IC)], idx_v)
      pltpu.sync_copy(srcT.at[pl.ds(c0, CW), pl.ds(i0, IC)], src_v)
      for c in range(CW):
        addc = base + l0 + c

        @pl.loop(0, V, unroll=8)
        def _vregs(v):
          s = pl.ds(v * L, L)
          d = pl.ds(c * IC + v * L, L)
          offs_v[d] = (idx_v[c, s] << SH) + addc
          upd_v[d] = src_v[c, s]

      pltpu.async_copy(upd_v, w.at[offs_v], sem).wait()

  return k


@jax.jit
def _run(inp, idx, src):
  M, D = inp.shape
  B = idx.shape[0]
  NC = 2
  CPS = D // NC
  inpS = jnp.transpose(inp.reshape(M, NC, CPS), (1, 0, 2)).reshape(NC * M * CPS)
  idxT = jnp.transpose(idx)
  srcT = jnp.transpose(src)
  w = _build(M, D, B)(inpS, idxT, srcT)
  stripes = w.reshape(NC, M, CPS)
  return jnp.concatenate([stripes[i] for i in range(NC)], axis=1)


def kernel(input, dim, index, src):
  del dim
  return _run(input, index, src)
